# baseline (speedup 1.0000x reference)
import functools

import jax
import jax.numpy as jnp
from jax import lax
from jax.experimental import pallas as pl
from jax.experimental.pallas import tpu as pltpu
from jax.experimental.pallas import tpu_sc as plsc

N_NODES = 10000
N_PAD = 10240
N_EDGES = 160000
D_X_IN = 256
D_X_OUT = 256
D_E_IN = 16
D_E_OUT = 64

NUM_SC = 2
NUM_SUBCORES = 16
EDGES_PER_CORE = N_EDGES // NUM_SC
CHUNK_E = 16000
NUM_CHUNKS = EDGES_PER_CORE // CHUNK_E
LANES = 16

_SC_PARAMS = pltpu.CompilerParams(
    use_tc_tiling_on_sc=False, needs_layout_passes=False)


N_ROWS = N_PAD // 128


def _sc_segment_sum_t(eaT, edge_index):
    mesh = plsc.VectorSubcoreMesh(core_axis_name="c", subcore_axis_name="s")
    part = jax.ShapeDtypeStruct((D_E_IN, N_ROWS, 128), jnp.float32)

    @functools.partial(
        pl.kernel,
        out_type=[part, part],
        mesh=mesh,
        compiler_params=_SC_PARAMS,
        scratch_types=[
            pltpu.VMEM((N_ROWS, 128), jnp.float32),
            pltpu.VMEM((CHUNK_E,), jnp.int32),
            pltpu.VMEM((CHUNK_E // 128, 128), jnp.float32),
            pltpu.VMEM((CHUNK_E,), jnp.int32),
            pltpu.VMEM((CHUNK_E // 128, 128), jnp.float32),
            pltpu.SemaphoreType.DMA,
            pltpu.SemaphoreType.DMA,
        ],
    )
    def seg_sum(eaT_hbm, ei_hbm, p0_hbm, p1_hbm, acc,
                idx0, val0, idx1, val1, sem0, sem1):
        cid = lax.axis_index("c")
        sid = lax.axis_index("s")
        base = cid * EDGES_PER_CORE
        bufs = ((idx0, val0, sem0), (idx1, val1, sem1))

        def start(c, buf):
            idx_v, val_v, sem = buf
            off = base + c * CHUNK_E
            h1 = pltpu.async_copy(ei_hbm.at[1, pl.ds(off, CHUNK_E)], idx_v,
                                  sem)
            h2 = pltpu.async_copy(
                eaT_hbm.at[sid, pl.ds(off // 128, CHUNK_E // 128)], val_v,
                sem)
            return h1, h2

        pending = start(0, bufs[0])

        @pl.loop(0, N_ROWS)
        def _(r):
            @pl.loop(0, 128, step=LANES)
            def _(i):
                acc[r, pl.ds(i, LANES)] = jnp.zeros((LANES,), jnp.float32)

        for c in range(NUM_CHUNKS):
            idx_v, val_v, _ = bufs[c % 2]
            pending[0].wait()
            pending[1].wait()
            if c + 1 < NUM_CHUNKS:
                pending = start(c + 1, bufs[(c + 1) % 2])

            @plsc.parallel_loop(0, CHUNK_E // 128, unroll=2)
            def _(r):
                for u in range(128 // LANES):
                    idx = idx_v[pl.ds(r * 128 + u * LANES, LANES)]
                    hi = lax.shift_right_logical(idx, 7)
                    lo = lax.bitwise_and(idx, 127)
                    plsc.addupdate_scatter(
                        acc, [hi, lo], val_v[r, pl.ds(u * LANES, LANES)])

        @pl.when(cid == 0)
        def _():
            pltpu.sync_copy(acc, p0_hbm.at[sid])

        @pl.when(cid == 1)
        def _():
            pltpu.sync_copy(acc, p1_hbm.at[sid])

    return seg_sum(eaT, edge_index)


FLAT_COLS = 16384
E_ROWS = N_EDGES // 128


def _flat_body(eat_ref, out_ref):
    out_ref[...] = jnp.reshape(eat_ref[...], (D_E_IN, FLAT_COLS // 128, 128))


def _flatten_eat(eaT):
    return pl.pallas_call(
        _flat_body,
        grid=(pl.cdiv(N_EDGES, FLAT_COLS),),
        in_specs=[pl.BlockSpec((D_E_IN, FLAT_COLS), lambda i: (0, i))],
        out_specs=pl.BlockSpec((D_E_IN, FLAT_COLS // 128, 128),
                               lambda i: (0, i, 0)),
        out_shape=jax.ShapeDtypeStruct((D_E_IN, E_ROWS, 128), jnp.float32),
    )(eaT)


ROW_BLOCK = 1024
GRID_M = pl.cdiv(N_NODES, ROW_BLOCK)


def _tc_xh_body(x_ref, wx_ref, bxt_ref, outt_ref):
    xht = lax.dot_general(wx_ref[...], x_ref[...], (((1,), (1,)), ((), ())),
                          preferred_element_type=jnp.float32)
    outt_ref[...] = xht + bxt_ref[...]


def _tc_xh(x, wx, bxt):
    return pl.pallas_call(
        _tc_xh_body,
        grid=(GRID_M,),
        in_specs=[
            pl.BlockSpec((ROW_BLOCK, D_X_IN), lambda i: (i, 0)),
            pl.BlockSpec((D_X_OUT, D_X_IN), lambda i: (0, 0)),
            pl.BlockSpec((D_X_OUT, 1), lambda i: (0, 0)),
        ],
        out_specs=pl.BlockSpec((D_X_OUT, ROW_BLOCK), lambda i: (0, i)),
        out_shape=jax.ShapeDtypeStruct((D_X_OUT + D_E_OUT, N_NODES),
                                       jnp.float32),
    )(x, wx, bxt)


def _tc_es_body(outt_in_ref, p0_ref, p1_ref, we_ref, bet_ref, outt_ref):
    del outt_in_ref
    s_t = jnp.reshape(p0_ref[...] + p1_ref[...], (D_E_IN, N_PAD))
    est = lax.dot_general(we_ref[...], s_t[:, :N_NODES],
                          (((1,), (0,)), ((), ())),
                          preferred_element_type=jnp.float32)
    outt_ref[...] = est + bet_ref[...]


def _tc_es(outt, p0t, p1t, we, bet):
    return pl.pallas_call(
        _tc_es_body,
        grid=(1,),
        in_specs=[
            pl.BlockSpec(memory_space=pltpu.MemorySpace.HBM),
            pl.BlockSpec((D_E_IN, N_ROWS, 128), lambda i: (0, 0, 0)),
            pl.BlockSpec((D_E_IN, N_ROWS, 128), lambda i: (0, 0, 0)),
            pl.BlockSpec((D_E_OUT, D_E_IN), lambda i: (0, 0)),
            pl.BlockSpec((D_E_OUT, 1), lambda i: (0, 0)),
        ],
        out_specs=pl.BlockSpec((D_E_OUT, N_NODES),
                               lambda i: (D_X_OUT // D_E_OUT, 0)),
        out_shape=jax.ShapeDtypeStruct((D_X_OUT + D_E_OUT, N_NODES),
                                       jnp.float32),
        input_output_aliases={0: 0},
    )(outt, p0t, p1t, we, bet)


def kernel(x, edge_index, edge_attr, Wx, bx, We, be):
    ea_flat = _flatten_eat(edge_attr.T)
    p0t, p1t = _sc_segment_sum_t(ea_flat, edge_index.astype(jnp.int32))
    outt = _tc_xh(x, Wx, bx.reshape(-1, 1))
    outt = _tc_es(outt, p0t, p1t, We, be.reshape(-1, 1))
    return outt.T


# speedup vs baseline: 1.1922x; 1.1922x over previous
import functools

import jax
import jax.numpy as jnp
from jax import lax
from jax.experimental import pallas as pl
from jax.experimental.pallas import tpu as pltpu
from jax.experimental.pallas import tpu_sc as plsc

N_NODES = 10000
N_PAD = 10240
N_EDGES = 160000
D_X_IN = 256
D_X_OUT = 256
D_E_IN = 16
D_E_OUT = 64

NUM_SC = 2
NUM_SUBCORES = 16
EDGES_PER_CORE = N_EDGES // NUM_SC
CHUNK_E = 16000
NUM_CHUNKS = EDGES_PER_CORE // CHUNK_E
LANES = 16

_SC_PARAMS = pltpu.CompilerParams(
    use_tc_tiling_on_sc=False, needs_layout_passes=False)


N_ROWS = N_PAD // 128


def _sc_segment_sum_t(eaT, edge_index):
    mesh = plsc.VectorSubcoreMesh(core_axis_name="c", subcore_axis_name="s")
    part = jax.ShapeDtypeStruct((D_E_IN, N_ROWS, 128), jnp.float32)

    @functools.partial(
        pl.kernel,
        out_type=[part, part],
        mesh=mesh,
        compiler_params=_SC_PARAMS,
        scratch_types=[
            pltpu.VMEM((N_ROWS, 128), jnp.float32),
            pltpu.VMEM((CHUNK_E,), jnp.int32),
            pltpu.VMEM((CHUNK_E,), jnp.float32),
            pltpu.VMEM((CHUNK_E,), jnp.int32),
            pltpu.VMEM((CHUNK_E,), jnp.float32),
            pltpu.SemaphoreType.DMA,
            pltpu.SemaphoreType.DMA,
        ],
    )
    def seg_sum(eaT_hbm, ei_hbm, p0_hbm, p1_hbm, acc,
                idx0, val0, idx1, val1, sem0, sem1):
        cid = lax.axis_index("c")
        sid = lax.axis_index("s")
        base = cid * EDGES_PER_CORE
        bufs = ((idx0, val0, sem0), (idx1, val1, sem1))

        def start(c, buf):
            idx_v, val_v, sem = buf
            off = base + c * CHUNK_E
            h1 = pltpu.async_copy(ei_hbm.at[1, pl.ds(off, CHUNK_E)], idx_v,
                                  sem)
            h2 = pltpu.async_copy(eaT_hbm.at[sid, pl.ds(off, CHUNK_E)],
                                  val_v, sem)
            return h1, h2

        pending = start(0, bufs[0])

        @pl.loop(0, N_ROWS)
        def _(r):
            @pl.loop(0, 128, step=LANES)
            def _(i):
                acc[r, pl.ds(i, LANES)] = jnp.zeros((LANES,), jnp.float32)

        for c in range(NUM_CHUNKS):
            idx_v, val_v, _ = bufs[c % 2]
            pending[0].wait()
            pending[1].wait()
            if c + 1 < NUM_CHUNKS:
                pending = start(c + 1, bufs[(c + 1) % 2])

            @plsc.parallel_loop(0, CHUNK_E, step=LANES, unroll=8)
            def _(i):
                idx = idx_v[pl.ds(i, LANES)]
                hi = lax.shift_right_logical(idx, 7)
                lo = lax.bitwise_and(idx, 127)
                plsc.addupdate_scatter(acc, [hi, lo], val_v[pl.ds(i, LANES)])

        @pl.when(cid == 0)
        def _():
            pltpu.sync_copy(acc, p0_hbm.at[sid])

        @pl.when(cid == 1)
        def _():
            pltpu.sync_copy(acc, p1_hbm.at[sid])

    return seg_sum(eaT, edge_index)


ROW_BLOCK = 1024
GRID_M = pl.cdiv(N_NODES, ROW_BLOCK)


def _tc_xh_body(x_ref, wx_ref, bxt_ref, outt_ref):
    xht = lax.dot_general(wx_ref[...], x_ref[...], (((1,), (1,)), ((), ())),
                          preferred_element_type=jnp.float32)
    outt_ref[...] = xht + bxt_ref[...]


def _tc_xh(x, wx, bxt):
    return pl.pallas_call(
        _tc_xh_body,
        grid=(GRID_M,),
        in_specs=[
            pl.BlockSpec((ROW_BLOCK, D_X_IN), lambda i: (i, 0)),
            pl.BlockSpec((D_X_OUT, D_X_IN), lambda i: (0, 0)),
            pl.BlockSpec((D_X_OUT, 1), lambda i: (0, 0)),
        ],
        out_specs=pl.BlockSpec((D_X_OUT, ROW_BLOCK), lambda i: (0, i)),
        out_shape=jax.ShapeDtypeStruct((D_X_OUT + D_E_OUT, N_NODES),
                                       jnp.float32),
    )(x, wx, bxt)


def _tc_es_body(outt_in_ref, p0_ref, p1_ref, we_ref, bet_ref, outt_ref):
    del outt_in_ref
    s_t = jnp.reshape(p0_ref[...] + p1_ref[...], (D_E_IN, N_PAD))
    est = lax.dot_general(we_ref[...], s_t[:, :N_NODES],
                          (((1,), (0,)), ((), ())),
                          preferred_element_type=jnp.float32)
    outt_ref[...] = est + bet_ref[...]


def _tc_es(outt, p0t, p1t, we, bet):
    return pl.pallas_call(
        _tc_es_body,
        grid=(1,),
        in_specs=[
            pl.BlockSpec(memory_space=pltpu.MemorySpace.HBM),
            pl.BlockSpec((D_E_IN, N_ROWS, 128), lambda i: (0, 0, 0)),
            pl.BlockSpec((D_E_IN, N_ROWS, 128), lambda i: (0, 0, 0)),
            pl.BlockSpec((D_E_OUT, D_E_IN), lambda i: (0, 0)),
            pl.BlockSpec((D_E_OUT, 1), lambda i: (0, 0)),
        ],
        out_specs=pl.BlockSpec((D_E_OUT, N_NODES),
                               lambda i: (D_X_OUT // D_E_OUT, 0)),
        out_shape=jax.ShapeDtypeStruct((D_X_OUT + D_E_OUT, N_NODES),
                                       jnp.float32),
        input_output_aliases={0: 0},
    )(outt, p0t, p1t, we, bet)


def kernel(x, edge_index, edge_attr, Wx, bx, We, be):
    eaT = edge_attr.T
    p0t, p1t = _sc_segment_sum_t(eaT, edge_index.astype(jnp.int32))
    outt = _tc_xh(x, Wx, bx.reshape(-1, 1))
    outt = _tc_es(outt, p0t, p1t, We, be.reshape(-1, 1))
    return outt.T
